# Initial kernel scaffold; baseline (speedup 1.0000x reference)
import jax
import jax.numpy as jnp
from jax.experimental import pallas as pl


def kernel(x, edge_index, W1, b1, Wl, bl, Wr, Wn):
    raise NotImplementedError("write your pallas kernel here")


# speedup vs baseline: 5.5154x; 5.5154x over previous
import functools

import jax
import jax.numpy as jnp
from jax import lax
from jax.experimental import pallas as pl
from jax.experimental.pallas import tpu as pltpu
from jax.experimental.pallas import tpu_sc as plsc

_N = 10000
_E = 320000
_XD = 128
_HID = 128
_NCLS = 40
_HP = 144

_NC = 2
_NS = 16
_NW = _NC * _NS
_EPW = _E // _NW
_K = 80
_NCH = _EPW // _K
_RPS = _N // _NS
_ZCH = 125

_BN = 1000


def _pre_body(x_ref, w1_ref, b1_ref, feat_ref, hp_ref):
    h = jnp.dot(x_ref[...], w1_ref[...], preferred_element_type=jnp.float32)
    h = h + b1_ref[...]
    feat_ref[...] = h
    hr = jnp.maximum(h, 0.0)
    hp_ref[:, :_HID] = hr
    tail = (lax.broadcasted_iota(jnp.int32, (_BN, _HP - _HID), 1) == 0)
    hp_ref[:, _HID:] = tail.astype(jnp.float32)


def _sc_agg_body(hp_hbm, src_hbm, dst_hbm, out_hbm,
                 src_v, dst_v, rows_v, acc_sh, sem):
    c = lax.axis_index("c")
    s = lax.axis_index("s")
    wid = c * _NS + s

    zv = jnp.zeros((16,), jnp.float32)

    @pl.loop(0, _ZCH)
    def _(r):
        @pl.loop(0, _HP, step=16)
        def _(c0):
            rows_v[r, pl.ds(c0, 16)] = zv

    @pl.loop(0, _RPS, step=_ZCH)
    def _(r0):
        pltpu.sync_copy(rows_v, acc_sh.at[pl.ds(s * _RPS + r0, _ZCH), :])

    plsc.subcore_barrier()

    base0 = wid * _EPW

    @pl.loop(0, _NCH)
    def _(j):
        base = base0 + j * _K
        pltpu.sync_copy(src_hbm.at[pl.ds(base, _K)], src_v)
        pltpu.sync_copy(dst_hbm.at[pl.ds(base, _K)], dst_v)
        gbuf = rows_v.at[pl.ds(0, _K), :]
        pltpu.async_copy(hp_hbm.at[src_v], gbuf, sem).wait()
        pltpu.sync_copy(gbuf, acc_sh.at[dst_v], add=True)

    plsc.subcore_barrier()

    pltpu.sync_copy(acc_sh.at[pl.ds(s * _RPS, _RPS), :],
                    out_hbm.at[c, pl.ds(s * _RPS, _RPS), :])


def _post_body(part_ref, hp_ref, wl_ref, bl_ref, wr_ref, wn_ref,
               out_ref, of_ref):
    ssum = part_ref[0] + part_ref[1]
    agg = ssum[:, :_HID]
    cnt = ssum[:, _HID:_HID + 1]
    mean = agg / jnp.maximum(cnt, 1.0)
    hr = hp_ref[:, :_HID]
    of = jnp.dot(mean, wl_ref[...], preferred_element_type=jnp.float32)
    of = of + bl_ref[...]
    of = of + jnp.dot(hr, wr_ref[...], preferred_element_type=jnp.float32)
    of_ref[...] = of
    nrm = jnp.sqrt(jnp.sum(of * of, axis=1, keepdims=True))
    xn = of / jnp.maximum(nrm, 1e-12)
    w = wn_ref[...]
    wnrm = jnp.sqrt(jnp.sum(w * w, axis=0, keepdims=True))
    wn = w / jnp.maximum(wnrm, 1e-12)
    out_ref[...] = 10.0 * jnp.dot(xn, wn, preferred_element_type=jnp.float32)


@jax.jit
def _run(x, src, dst, W1, b1, Wl, bl, Wr, Wn):
    feat, hp = pl.pallas_call(
        _pre_body,
        grid=(_N // _BN,),
        in_specs=[
            pl.BlockSpec((_BN, _XD), lambda i: (i, 0)),
            pl.BlockSpec((_XD, _HID), lambda i: (0, 0)),
            pl.BlockSpec((1, _HID), lambda i: (0, 0)),
        ],
        out_specs=[
            pl.BlockSpec((_BN, _HID), lambda i: (i, 0)),
            pl.BlockSpec((_BN, _HP), lambda i: (i, 0)),
        ],
        out_shape=[
            jax.ShapeDtypeStruct((_N, _HID), jnp.float32),
            jax.ShapeDtypeStruct((_N, _HP), jnp.float32),
        ],
    )(x, W1, b1.reshape(1, _HID))

    mesh = plsc.VectorSubcoreMesh(core_axis_name="c", subcore_axis_name="s")
    partials = pl.kernel(
        _sc_agg_body,
        out_type=jax.ShapeDtypeStruct((_NC, _N, _HP), jnp.float32),
        mesh=mesh,
        compiler_params=pltpu.CompilerParams(use_tc_tiling_on_sc=False),
        scratch_types=[
            pltpu.VMEM((_K,), jnp.int32),
            pltpu.VMEM((_K,), jnp.int32),
            pltpu.VMEM((_ZCH, _HP), jnp.float32),
            pltpu.VMEM_SHARED((_N, _HP), jnp.float32),
            pltpu.SemaphoreType.DMA,
        ],
    )(hp, src, dst)

    out, out_feat = pl.pallas_call(
        _post_body,
        grid=(_N // _BN,),
        in_specs=[
            pl.BlockSpec((_NC, _BN, _HP), lambda i: (0, i, 0)),
            pl.BlockSpec((_BN, _HP), lambda i: (i, 0)),
            pl.BlockSpec((_HID, _HID), lambda i: (0, 0)),
            pl.BlockSpec((1, _HID), lambda i: (0, 0)),
            pl.BlockSpec((_HID, _HID), lambda i: (0, 0)),
            pl.BlockSpec((_HID, _NCLS), lambda i: (0, 0)),
        ],
        out_specs=[
            pl.BlockSpec((_BN, _NCLS), lambda i: (i, 0)),
            pl.BlockSpec((_BN, _HID), lambda i: (i, 0)),
        ],
        out_shape=[
            jax.ShapeDtypeStruct((_N, _NCLS), jnp.float32),
            jax.ShapeDtypeStruct((_N, _HID), jnp.float32),
        ],
    )(partials, hp, Wl, bl.reshape(1, _HID), Wr, Wn)

    return out, feat, out_feat


def kernel(x, edge_index, W1, b1, Wl, bl, Wr, Wn):
    src = edge_index[0].astype(jnp.int32)
    dst = edge_index[1].astype(jnp.int32)
    return _run(x, src, dst, W1, b1, Wl, bl, Wr, Wn)


# speedup vs baseline: 8.7718x; 1.5904x over previous
import functools

import jax
import jax.numpy as jnp
from jax import lax
from jax.experimental import pallas as pl
from jax.experimental.pallas import tpu as pltpu
from jax.experimental.pallas import tpu_sc as plsc

_N = 10000
_E = 320000
_XD = 128
_HID = 128
_NCLS = 40
_HP = 144

_NC = 2
_NS = 16
_NW = _NC * _NS
_EPW = _E // _NW
_K = 80
_NCH = _EPW // _K
_RPS = _N // _NS
_ZCH = 125

_BN = 1000


def _pre_body(x_ref, w1_ref, b1_ref, feat_ref, hp_ref):
    h = jnp.dot(x_ref[...], w1_ref[...], preferred_element_type=jnp.float32)
    h = h + b1_ref[...]
    feat_ref[...] = h
    hr = jnp.maximum(h, 0.0)
    hp_ref[:, :_HID] = hr
    tail = (lax.broadcasted_iota(jnp.int32, (_BN, _HP - _HID), 1) == 0)
    hp_ref[:, _HID:] = tail.astype(jnp.float32)


def _sc_agg_body(idx_hbm, hp_hbm, out_hbm,
                 ib0, ib1, rows0, rows1, acc_sh, isem0, isem1, gsem0, gsem1):
    c = lax.axis_index("c")
    s = lax.axis_index("s")
    wid = c * _NS + s

    zv = jnp.zeros((16,), jnp.float32)

    @pl.loop(0, _K)
    def _(r):
        @pl.loop(0, _HP, step=16)
        def _(c0):
            rows0[r, pl.ds(c0, 16)] = zv

    @pl.loop(0, _RPS - _K, step=_K)
    def _(r0):
        pltpu.sync_copy(rows0, acc_sh.at[pl.ds(s * _RPS + r0, _K), :])

    tail = _RPS - (_RPS // _K) * _K
    pltpu.sync_copy(rows0.at[pl.ds(0, tail), :],
                    acc_sh.at[pl.ds(s * _RPS + _RPS - tail, tail), :])

    plsc.subcore_barrier()

    def load_idx(j, ib, sem):
        pltpu.async_copy(idx_hbm.at[wid, j], ib, sem)

    def wait_idx(ib, sem):
        pltpu.make_async_copy(idx_hbm.at[0, 0], ib, sem).wait()

    def start_gather(ib, buf, sem):
        pltpu.async_copy(hp_hbm.at[ib.at[0]], buf, sem)

    def wait_gather(buf, sem):
        pltpu.make_async_copy(hp_hbm.at[ib0.at[0]], buf, sem).wait()

    def scatter_add(buf, ib):
        pltpu.sync_copy(buf, acc_sh.at[ib.at[1]], add=True)

    pltpu.async_copy(idx_hbm.at[wid, 0], ib0, isem0)
    wait_idx(ib0, isem0)
    start_gather(ib0, rows0, gsem0)
    load_idx(1, ib1, isem1)

    @pl.loop(0, (_NCH - 1) // 2)
    def _(j2):
        a = 2 * j2
        wait_gather(rows0, gsem0)
        wait_idx(ib1, isem1)
        start_gather(ib1, rows1, gsem1)
        scatter_add(rows0, ib0)
        load_idx(a + 2, ib0, isem0)
        wait_gather(rows1, gsem1)
        wait_idx(ib0, isem0)
        start_gather(ib0, rows0, gsem0)
        scatter_add(rows1, ib1)
        load_idx(a + 3, ib1, isem1)

    wait_gather(rows0, gsem0)
    scatter_add(rows0, ib0)
    wait_idx(ib1, isem1)

    plsc.subcore_barrier()

    pltpu.sync_copy(acc_sh.at[pl.ds(s * _RPS, _RPS), :],
                    out_hbm.at[c, pl.ds(s * _RPS, _RPS), :])


def _post_body(part_ref, hp_ref, wl_ref, bl_ref, wr_ref, wn_ref,
               out_ref, of_ref):
    ssum = part_ref[0] + part_ref[1]
    agg = ssum[:, :_HID]
    cnt = ssum[:, _HID:_HID + 1]
    mean = agg / jnp.maximum(cnt, 1.0)
    hr = hp_ref[:, :_HID]
    of = jnp.dot(mean, wl_ref[...], preferred_element_type=jnp.float32)
    of = of + bl_ref[...]
    of = of + jnp.dot(hr, wr_ref[...], preferred_element_type=jnp.float32)
    of_ref[...] = of
    nrm = jnp.sqrt(jnp.sum(of * of, axis=1, keepdims=True))
    xn = of / jnp.maximum(nrm, 1e-12)
    w = wn_ref[...]
    wnrm = jnp.sqrt(jnp.sum(w * w, axis=0, keepdims=True))
    wn = w / jnp.maximum(wnrm, 1e-12)
    out_ref[...] = 10.0 * jnp.dot(xn, wn, preferred_element_type=jnp.float32)


@jax.jit
def _run(x, idx, W1, b1, Wl, bl, Wr, Wn):
    feat, hp = pl.pallas_call(
        _pre_body,
        grid=(_N // _BN,),
        in_specs=[
            pl.BlockSpec((_BN, _XD), lambda i: (i, 0)),
            pl.BlockSpec((_XD, _HID), lambda i: (0, 0)),
            pl.BlockSpec((1, _HID), lambda i: (0, 0)),
        ],
        out_specs=[
            pl.BlockSpec((_BN, _HID), lambda i: (i, 0)),
            pl.BlockSpec((_BN, _HP), lambda i: (i, 0)),
        ],
        out_shape=[
            jax.ShapeDtypeStruct((_N, _HID), jnp.float32),
            jax.ShapeDtypeStruct((_N, _HP), jnp.float32),
        ],
    )(x, W1, b1.reshape(1, _HID))

    mesh = plsc.VectorSubcoreMesh(core_axis_name="c", subcore_axis_name="s")
    partials = pl.kernel(
        _sc_agg_body,
        out_type=jax.ShapeDtypeStruct((_NC, _N, _HP), jnp.float32),
        mesh=mesh,
        compiler_params=pltpu.CompilerParams(use_tc_tiling_on_sc=False),
        scratch_types=[
            pltpu.VMEM((2, _K), jnp.int32),
            pltpu.VMEM((2, _K), jnp.int32),
            pltpu.VMEM((_K, _HP), jnp.float32),
            pltpu.VMEM((_K, _HP), jnp.float32),
            pltpu.VMEM_SHARED((_N, _HP), jnp.float32),
            pltpu.SemaphoreType.DMA,
            pltpu.SemaphoreType.DMA,
            pltpu.SemaphoreType.DMA,
            pltpu.SemaphoreType.DMA,
        ],
    )(idx, hp)

    out, out_feat = pl.pallas_call(
        _post_body,
        grid=(_N // _BN,),
        in_specs=[
            pl.BlockSpec((_NC, _BN, _HP), lambda i: (0, i, 0)),
            pl.BlockSpec((_BN, _HP), lambda i: (i, 0)),
            pl.BlockSpec((_HID, _HID), lambda i: (0, 0)),
            pl.BlockSpec((1, _HID), lambda i: (0, 0)),
            pl.BlockSpec((_HID, _HID), lambda i: (0, 0)),
            pl.BlockSpec((_HID, _NCLS), lambda i: (0, 0)),
        ],
        out_specs=[
            pl.BlockSpec((_BN, _NCLS), lambda i: (i, 0)),
            pl.BlockSpec((_BN, _HID), lambda i: (i, 0)),
        ],
        out_shape=[
            jax.ShapeDtypeStruct((_N, _NCLS), jnp.float32),
            jax.ShapeDtypeStruct((_N, _HID), jnp.float32),
        ],
    )(partials, hp, Wl, bl.reshape(1, _HID), Wr, Wn)

    return out, feat, out_feat


def kernel(x, edge_index, W1, b1, Wl, bl, Wr, Wn):
    sd = edge_index.astype(jnp.int32).reshape(2, _NW, _NCH, _K)
    sd = jnp.moveaxis(sd, 0, 2)
    idx = jnp.concatenate(
        [sd, jnp.zeros((_NW, 1, 2, _K), jnp.int32)], axis=1)
    return _run(x, idx, W1, b1, Wl, bl, Wr, Wn)


# speedup vs baseline: 9.3125x; 1.0616x over previous
import functools

import jax
import jax.numpy as jnp
from jax import lax
from jax.experimental import pallas as pl
from jax.experimental.pallas import tpu as pltpu
from jax.experimental.pallas import tpu_sc as plsc

_N = 10000
_E = 320000
_XD = 128
_HID = 128
_NCLS = 40
_HP = 144

_NC = 2
_NS = 16
_NW = _NC * _NS
_EPW = _E // _NW
_K = 80
_NCH = _EPW // _K
_RPS = _N // _NS
_ZCH = 125

_BN = 1000


def _pre_body(x_ref, w1_ref, b1_ref, feat_ref, hp_ref):
    h = jnp.dot(x_ref[...], w1_ref[...], preferred_element_type=jnp.float32)
    h = h + b1_ref[...]
    feat_ref[...] = h
    hr = jnp.maximum(h, 0.0)
    hp_ref[:, :_HID] = hr
    tail = (lax.broadcasted_iota(jnp.int32, (_BN, _HP - _HID), 1) == 0)
    hp_ref[:, _HID:] = tail.astype(jnp.float32)


def _sc_agg_body(e_hbm, hp_hbm, out_hbm,
                 ib0, ib1, ib2, rows0, rows1, rows2, acc_sh,
                 i0, i1, i2, g0, g1, g2, s0, s1, s2):
    c = lax.axis_index("c")
    s = lax.axis_index("s")
    wid = c * _NS + s
    base0 = wid * _EPW

    zv = jnp.zeros((16,), jnp.float32)

    @pl.loop(0, _K)
    def _(r):
        @pl.loop(0, _HP, step=16)
        def _(c0):
            rows0[r, pl.ds(c0, 16)] = zv

    @pl.loop(0, _RPS - _K, step=_K)
    def _(r0):
        pltpu.sync_copy(rows0, acc_sh.at[pl.ds(s * _RPS + r0, _K), :])

    tail = _RPS - (_RPS // _K) * _K
    pltpu.sync_copy(rows0.at[pl.ds(0, tail), :],
                    acc_sh.at[pl.ds(s * _RPS + _RPS - tail, tail), :])

    plsc.subcore_barrier()

    ibs = (ib0, ib1, ib2)
    bufs = (rows0, rows1, rows2)
    isems = (i0, i1, i2)
    gsems = (g0, g1, g2)
    ssems = (s0, s1, s2)

    def load_idx(t, ib, sem):
        off = pl.multiple_of(base0 + jnp.minimum(t, _NCH - 1) * _K, 8)
        pltpu.async_copy(e_hbm.at[0, pl.ds(off, _K)], ib.at[0], sem)
        pltpu.async_copy(e_hbm.at[1, pl.ds(off, _K)], ib.at[1], sem)

    def wait_idx(ib, sem):
        pltpu.make_async_copy(e_hbm.at[:, pl.ds(0, _K)], ib, sem).wait()

    def start_g(ib, buf, sem):
        pltpu.async_copy(hp_hbm.at[ib.at[0]], buf, sem)

    def wait_g(buf, sem):
        pltpu.make_async_copy(hp_hbm.at[pl.ds(0, _K), :], buf, sem).wait()

    def start_sc(buf, ib, sem):
        pltpu.async_copy(buf, acc_sh.at[ib.at[1]], sem, add=True)

    def wait_sc(buf, sem):
        pltpu.make_async_copy(buf, acc_sh.at[pl.ds(0, _K), :], sem).wait()

    def step(t, sl, first=False, last=False):
        r, r1, r2 = sl, (sl + 1) % 3, (sl + 2) % 3
        if not last:
            wait_idx(ibs[r1], isems[r1])
        if not first:
            wait_sc(bufs[r2], ssems[r2])
        wait_g(bufs[r], gsems[r])
        if not last:
            start_g(ibs[r1], bufs[r1], gsems[r1])
        start_sc(bufs[r], ibs[r], ssems[r])
        if not last:
            load_idx(t + 2, ibs[r2], isems[r2])

    load_idx(0, ib0, i0)
    wait_idx(ib0, i0)
    load_idx(1, ib1, i1)
    start_g(ib0, rows0, g0)
    step(jnp.int32(0), 0, first=True)

    @pl.loop(0, (_NCH - 2) // 3)
    def _(j3):
        t = 3 * j3 + 1
        step(t, 1)
        step(t + 1, 2)
        step(t + 2, 0)

    wait_idx(ibs[2], isems[2])
    wait_sc(bufs[0], ssems[0])
    wait_g(bufs[1], gsems[1])
    start_sc(bufs[1], ibs[1], ssems[1])
    wait_sc(bufs[1], ssems[1])

    plsc.subcore_barrier()

    pltpu.sync_copy(acc_sh.at[pl.ds(s * _RPS, _RPS), :],
                    out_hbm.at[c, pl.ds(s * _RPS, _RPS), :])


def _post_body(part_ref, hp_ref, wl_ref, bl_ref, wr_ref, wn_ref,
               out_ref, of_ref):
    ssum = part_ref[0] + part_ref[1]
    agg = ssum[:, :_HID]
    cnt = ssum[:, _HID:_HID + 1]
    mean = agg / jnp.maximum(cnt, 1.0)
    hr = hp_ref[:, :_HID]
    of = jnp.dot(mean, wl_ref[...], preferred_element_type=jnp.float32)
    of = of + bl_ref[...]
    of = of + jnp.dot(hr, wr_ref[...], preferred_element_type=jnp.float32)
    of_ref[...] = of
    nrm = jnp.sqrt(jnp.sum(of * of, axis=1, keepdims=True))
    xn = of / jnp.maximum(nrm, 1e-12)
    w = wn_ref[...]
    wnrm = jnp.sqrt(jnp.sum(w * w, axis=0, keepdims=True))
    wn = w / jnp.maximum(wnrm, 1e-12)
    out_ref[...] = 10.0 * jnp.dot(xn, wn, preferred_element_type=jnp.float32)


@jax.jit
def _run(x, e, W1, b1, Wl, bl, Wr, Wn):
    feat, hp = pl.pallas_call(
        _pre_body,
        grid=(_N // _BN,),
        in_specs=[
            pl.BlockSpec((_BN, _XD), lambda i: (i, 0)),
            pl.BlockSpec((_XD, _HID), lambda i: (0, 0)),
            pl.BlockSpec((1, _HID), lambda i: (0, 0)),
        ],
        out_specs=[
            pl.BlockSpec((_BN, _HID), lambda i: (i, 0)),
            pl.BlockSpec((_BN, _HP), lambda i: (i, 0)),
        ],
        out_shape=[
            jax.ShapeDtypeStruct((_N, _HID), jnp.float32),
            jax.ShapeDtypeStruct((_N, _HP), jnp.float32),
        ],
    )(x, W1, b1.reshape(1, _HID))

    mesh = plsc.VectorSubcoreMesh(core_axis_name="c", subcore_axis_name="s")
    partials = pl.kernel(
        _sc_agg_body,
        out_type=jax.ShapeDtypeStruct((_NC, _N, _HP), jnp.float32),
        mesh=mesh,
        compiler_params=pltpu.CompilerParams(use_tc_tiling_on_sc=False),
        scratch_types=[
            pltpu.VMEM((2, _K), jnp.int32),
            pltpu.VMEM((2, _K), jnp.int32),
            pltpu.VMEM((2, _K), jnp.int32),
            pltpu.VMEM((_K, _HP), jnp.float32),
            pltpu.VMEM((_K, _HP), jnp.float32),
            pltpu.VMEM((_K, _HP), jnp.float32),
            pltpu.VMEM_SHARED((_N, _HP), jnp.float32),
        ] + [pltpu.SemaphoreType.DMA] * 9,
    )(e, hp)

    out, out_feat = pl.pallas_call(
        _post_body,
        grid=(_N // _BN,),
        in_specs=[
            pl.BlockSpec((_NC, _BN, _HP), lambda i: (0, i, 0)),
            pl.BlockSpec((_BN, _HP), lambda i: (i, 0)),
            pl.BlockSpec((_HID, _HID), lambda i: (0, 0)),
            pl.BlockSpec((1, _HID), lambda i: (0, 0)),
            pl.BlockSpec((_HID, _HID), lambda i: (0, 0)),
            pl.BlockSpec((_HID, _NCLS), lambda i: (0, 0)),
        ],
        out_specs=[
            pl.BlockSpec((_BN, _NCLS), lambda i: (i, 0)),
            pl.BlockSpec((_BN, _HID), lambda i: (i, 0)),
        ],
        out_shape=[
            jax.ShapeDtypeStruct((_N, _NCLS), jnp.float32),
            jax.ShapeDtypeStruct((_N, _HID), jnp.float32),
        ],
    )(partials, hp, Wl, bl.reshape(1, _HID), Wr, Wn)

    return out, feat, out_feat


def kernel(x, edge_index, W1, b1, Wl, bl, Wr, Wn):
    return _run(x, edge_index.astype(jnp.int32), W1, b1, Wl, bl, Wr, Wn)


# speedup vs baseline: 9.4844x; 1.0185x over previous
import functools

import jax
import jax.numpy as jnp
from jax import lax
from jax.experimental import pallas as pl
from jax.experimental.pallas import tpu as pltpu
from jax.experimental.pallas import tpu_sc as plsc

_N = 10000
_E = 320000
_XD = 128
_HID = 128
_NCLS = 40

_NC = 2
_NS = 16
_NW = _NC * _NS
_EPW = _E // _NW
_K = 80
_NCH = _EPW // _K
_RPS = _N // _NS
_ZCH = 125

_RBIG = 640
_RLAST = _N - 15 * _RBIG

_BN = 1000


def _pre_body(x_ref, w1_ref, b1_ref, feat_ref, hp_ref):
    h = jnp.dot(x_ref[...], w1_ref[...], preferred_element_type=jnp.float32)
    h = h + b1_ref[...]
    feat_ref[...] = h
    hp_ref[...] = jnp.maximum(h, 0.0)


def _sc_cnt_body(src_hbm, dst_hbm, out_hbm,
                 ones_v, zbuf, ib0, ib1, ib2, acc_sh, i0, i1, i2, s0, s1, s2):
    c = lax.axis_index("c")
    s = lax.axis_index("s")
    wid = c * _NS + s
    base0 = wid * _EPW

    ov = jnp.ones((16,), jnp.float32)
    zv = jnp.zeros((16,), jnp.float32)

    @pl.loop(0, _K)
    def _(r):
        ones_v[r, pl.ds(0, 16)] = ov

    @pl.loop(0, _ZCH)
    def _(r):
        zbuf[r, pl.ds(0, 16)] = zv

    @pl.loop(0, _RPS, step=_ZCH)
    def _(r0):
        pltpu.sync_copy(zbuf, acc_sh.at[pl.ds(s * _RPS + r0, _ZCH), :])

    plsc.subcore_barrier()

    ibs = (ib0, ib1, ib2)
    isems = (i0, i1, i2)
    ssems = (s0, s1, s2)

    def load_idx(t, ib, sem):
        off = pl.multiple_of(base0 + jnp.minimum(t, _NCH - 1) * _K, 8)
        pltpu.async_copy(dst_hbm.at[pl.ds(off, _K)], ib, sem)

    def wait_idx(ib, sem):
        pltpu.make_async_copy(dst_hbm.at[pl.ds(0, _K)], ib, sem).wait()

    def start_sc(ib, sem):
        pltpu.async_copy(ones_v, acc_sh.at[ib], sem, add=True)

    def wait_sc(sem):
        pltpu.make_async_copy(ones_v, acc_sh.at[pl.ds(0, _K), :], sem).wait()

    def step(t, sl, first=False, last=False):
        r, r1, r2 = sl, (sl + 1) % 3, (sl + 2) % 3
        wait_idx(ibs[r], isems[r])
        start_sc(ibs[r], ssems[r])
        if not first:
            wait_sc(ssems[r2])
        if not last:
            load_idx(t + 2, ibs[r2], isems[r2])

    load_idx(0, ib0, i0)
    load_idx(1, ib1, i1)
    step(jnp.int32(0), 0, first=True)

    @pl.loop(0, (_NCH - 2) // 3)
    def _(j3):
        t = 3 * j3 + 1
        step(t, 1)
        step(t + 1, 2)
        step(t + 2, 0)

    wait_idx(ibs[1], isems[1])
    start_sc(ibs[1], ssems[1])
    wait_sc(ssems[0])
    wait_idx(ibs[2], isems[2])
    wait_sc(ssems[1])

    plsc.subcore_barrier()

    pltpu.sync_copy(acc_sh.at[pl.ds(s * _RPS, _RPS), :],
                    out_hbm.at[c, pl.ds(s * _RPS, _RPS), :])


def _sc_agg_body(src_hbm, dst_hbm, hp_hbm, out_hbm,
                 sb0, sb1, sb2, db0, db1, db2, rows0, rows1, rows2, acc_sh,
                 i0, i1, i2, g0, g1, g2, s0, s1, s2):
    c = lax.axis_index("c")
    s = lax.axis_index("s")
    wid = c * _NS + s
    base0 = wid * _EPW

    row0 = s * _RBIG
    zv = jnp.zeros((16,), jnp.float32)

    @pl.loop(0, _K)
    def _(r):
        @pl.loop(0, _HID, step=16)
        def _(c0):
            rows0[r, pl.ds(c0, 16)] = zv

    @pl.when(s < 15)
    def _():
        @pl.loop(0, _RBIG, step=_K)
        def _(r0):
            pltpu.sync_copy(rows0, acc_sh.at[pl.ds(row0 + r0, _K), :])

    @pl.when(s == 15)
    def _():
        @pl.loop(0, _RLAST, step=_K)
        def _(r0):
            pltpu.sync_copy(rows0, acc_sh.at[pl.ds(row0 + r0, _K), :])

    plsc.subcore_barrier()

    sbs = (sb0, sb1, sb2)
    dbs = (db0, db1, db2)
    bufs = (rows0, rows1, rows2)
    isems = (i0, i1, i2)
    gsems = (g0, g1, g2)
    ssems = (s0, s1, s2)

    def load_idx(t, sb, db, sem):
        off = pl.multiple_of(base0 + jnp.minimum(t, _NCH - 1) * _K, 8)
        pltpu.async_copy(src_hbm.at[pl.ds(off, _K)], sb, sem)
        pltpu.async_copy(dst_hbm.at[pl.ds(off, _K)], db, sem)

    def wait_idx(sb, db, sem):
        pltpu.make_async_copy(src_hbm.at[pl.ds(0, _K)], sb, sem).wait()
        pltpu.make_async_copy(dst_hbm.at[pl.ds(0, _K)], db, sem).wait()

    def start_g(sb, buf, sem):
        pltpu.async_copy(hp_hbm.at[sb], buf, sem)

    def wait_g(buf, sem):
        pltpu.make_async_copy(hp_hbm.at[pl.ds(0, _K), :], buf, sem).wait()

    def start_sc(buf, db, sem):
        pltpu.async_copy(buf, acc_sh.at[db], sem, add=True)

    def wait_sc(buf, sem):
        pltpu.make_async_copy(buf, acc_sh.at[pl.ds(0, _K), :], sem).wait()

    def step(t, sl, first=False, last=False):
        r, r1, r2 = sl, (sl + 1) % 3, (sl + 2) % 3
        if not last:
            wait_idx(sbs[r1], dbs[r1], isems[r1])
        if not first:
            wait_sc(bufs[r2], ssems[r2])
        wait_g(bufs[r], gsems[r])
        if not last:
            start_g(sbs[r1], bufs[r1], gsems[r1])
        start_sc(bufs[r], dbs[r], ssems[r])
        if not last:
            load_idx(t + 2, sbs[r2], dbs[r2], isems[r2])

    load_idx(0, sb0, db0, i0)
    wait_idx(sb0, db0, i0)
    load_idx(1, sb1, db1, i1)
    start_g(sb0, rows0, g0)
    step(jnp.int32(0), 0, first=True)

    @pl.loop(0, (_NCH - 2) // 3)
    def _(j3):
        t = 3 * j3 + 1
        step(t, 1)
        step(t + 1, 2)
        step(t + 2, 0)

    wait_idx(sbs[2], dbs[2], isems[2])
    wait_sc(bufs[0], ssems[0])
    wait_g(bufs[1], gsems[1])
    start_sc(bufs[1], dbs[1], ssems[1])
    wait_sc(bufs[1], ssems[1])

    plsc.subcore_barrier()

    @pl.when(s < 15)
    def _():
        pltpu.sync_copy(acc_sh.at[pl.ds(row0, _RBIG), :],
                        out_hbm.at[c, pl.ds(row0, _RBIG), :])

    @pl.when(s == 15)
    def _():
        pltpu.sync_copy(acc_sh.at[pl.ds(row0, _RLAST), :],
                        out_hbm.at[c, pl.ds(row0, _RLAST), :])


def _post_body(part_ref, cnt_ref, hp_ref, wl_ref, bl_ref, wr_ref, wn_ref,
               out_ref, of_ref):
    agg = part_ref[0] + part_ref[1]
    cnt = cnt_ref[0, :, :1] + cnt_ref[1, :, :1]
    mean = agg / jnp.maximum(cnt, 1.0)
    hr = hp_ref[...]
    of = jnp.dot(mean, wl_ref[...], preferred_element_type=jnp.float32)
    of = of + bl_ref[...]
    of = of + jnp.dot(hr, wr_ref[...], preferred_element_type=jnp.float32)
    of_ref[...] = of
    nrm = jnp.sqrt(jnp.sum(of * of, axis=1, keepdims=True))
    xn = of / jnp.maximum(nrm, 1e-12)
    w = wn_ref[...]
    wnrm = jnp.sqrt(jnp.sum(w * w, axis=0, keepdims=True))
    wn = w / jnp.maximum(wnrm, 1e-12)
    out_ref[...] = 10.0 * jnp.dot(xn, wn, preferred_element_type=jnp.float32)


@jax.jit
def _run(x, src, dst, W1, b1, Wl, bl, Wr, Wn):
    mesh = plsc.VectorSubcoreMesh(core_axis_name="c", subcore_axis_name="s")

    cnt = pl.kernel(
        _sc_cnt_body,
        out_type=jax.ShapeDtypeStruct((_NC, _N, 16), jnp.float32),
        mesh=mesh,
        compiler_params=pltpu.CompilerParams(use_tc_tiling_on_sc=False),
        scratch_types=[
            pltpu.VMEM((_K, 16), jnp.float32),
            pltpu.VMEM((_ZCH, 16), jnp.float32),
            pltpu.VMEM((_K,), jnp.int32),
            pltpu.VMEM((_K,), jnp.int32),
            pltpu.VMEM((_K,), jnp.int32),
            pltpu.VMEM_SHARED((_N, 16), jnp.float32),
        ] + [pltpu.SemaphoreType.DMA] * 6,
    )(src, dst)

    feat, hp = pl.pallas_call(
        _pre_body,
        grid=(_N // _BN,),
        in_specs=[
            pl.BlockSpec((_BN, _XD), lambda i: (i, 0)),
            pl.BlockSpec((_XD, _HID), lambda i: (0, 0)),
            pl.BlockSpec((1, _HID), lambda i: (0, 0)),
        ],
        out_specs=[
            pl.BlockSpec((_BN, _HID), lambda i: (i, 0)),
            pl.BlockSpec((_BN, _HID), lambda i: (i, 0)),
        ],
        out_shape=[
            jax.ShapeDtypeStruct((_N, _HID), jnp.float32),
            jax.ShapeDtypeStruct((_N, _HID), jnp.float32),
        ],
    )(x, W1, b1.reshape(1, _HID))

    partials = pl.kernel(
        _sc_agg_body,
        out_type=jax.ShapeDtypeStruct((_NC, _N, _HID), jnp.float32),
        mesh=mesh,
        compiler_params=pltpu.CompilerParams(use_tc_tiling_on_sc=True),
        scratch_types=[
            pltpu.VMEM((_K,), jnp.int32),
            pltpu.VMEM((_K,), jnp.int32),
            pltpu.VMEM((_K,), jnp.int32),
            pltpu.VMEM((_K,), jnp.int32),
            pltpu.VMEM((_K,), jnp.int32),
            pltpu.VMEM((_K,), jnp.int32),
            pltpu.VMEM((_K, _HID), jnp.float32),
            pltpu.VMEM((_K, _HID), jnp.float32),
            pltpu.VMEM((_K, _HID), jnp.float32),
            pltpu.VMEM_SHARED((_N, _HID), jnp.float32),
        ] + [pltpu.SemaphoreType.DMA] * 9,
    )(src, dst, hp)

    out, out_feat = pl.pallas_call(
        _post_body,
        grid=(_N // _BN,),
        in_specs=[
            pl.BlockSpec((_NC, _BN, _HID), lambda i: (0, i, 0)),
            pl.BlockSpec((_NC, _BN, 16), lambda i: (0, i, 0)),
            pl.BlockSpec((_BN, _HID), lambda i: (i, 0)),
            pl.BlockSpec((_HID, _HID), lambda i: (0, 0)),
            pl.BlockSpec((1, _HID), lambda i: (0, 0)),
            pl.BlockSpec((_HID, _HID), lambda i: (0, 0)),
            pl.BlockSpec((_HID, _NCLS), lambda i: (0, 0)),
        ],
        out_specs=[
            pl.BlockSpec((_BN, _NCLS), lambda i: (i, 0)),
            pl.BlockSpec((_BN, _HID), lambda i: (i, 0)),
        ],
        out_shape=[
            jax.ShapeDtypeStruct((_N, _NCLS), jnp.float32),
            jax.ShapeDtypeStruct((_N, _HID), jnp.float32),
        ],
    )(partials, cnt, hp, Wl, bl.reshape(1, _HID), Wr, Wn)

    return out, feat, out_feat


def kernel(x, edge_index, W1, b1, Wl, bl, Wr, Wn):
    e = edge_index.astype(jnp.int32)
    return _run(x, e[0], e[1], W1, b1, Wl, bl, Wr, Wn)
